# baseline (speedup 1.0000x reference)
import functools
import jax
import jax.numpy as jnp
from jax import lax
from jax.experimental import pallas as pl
from jax.experimental.pallas import tpu as pltpu
from jax.experimental.pallas import tpu_sc as plsc

F = 1262
FP = 1280
N = 10000
NP = 10240
E = 40000
EP = 40960
NG = 16
RB = 256


def _worker_id():
    return lax.axis_index("s") * 2 + lax.axis_index("c")


NW = 32
EBK = 32
FCH = 128
BIG = 3.0e38


def _pad2(a, r, c):
    return jnp.pad(a, ((0, r - a.shape[0]), (0, c - a.shape[1])))


def _pad1(a, n):
    return jnp.pad(a, ((0, n - a.shape[0]),))


def _dot(a, b):
    return jnp.dot(a, b, preferred_element_type=jnp.float32)


def _node_body(x_ref, w1, b1, wd, b0, ws, x1_out, p_out, q_out):
    x1 = jnp.maximum(_dot(x_ref[...], w1[...]) + b1[...], 0.0)
    x1_out[...] = x1
    p_out[...] = _dot(x1, wd[...]) + b0[...]
    q_out[...] = _dot(x1, ws[...])


def _node_stage(xp, w1, b1, wd, b0, ws):
    nblk = NP // RB
    full = pl.BlockSpec((FP, FP), lambda i: (0, 0))
    brow = pl.BlockSpec((1, FP), lambda i: (0, 0))
    blk = pl.BlockSpec((RB, FP), lambda i: (i, 0))
    return pl.pallas_call(
        _node_body,
        grid=(nblk,),
        in_specs=[blk, full, brow, full, brow, full],
        out_specs=[blk, blk, blk],
        out_shape=[jax.ShapeDtypeStruct((NP, FP), jnp.float32)] * 3,
    )(xp, w1, b1, wd, b0, ws)


def _edge_mlp_body(g_ref, w1, b1, w2, b2, w3, b3, w4, b4, h_out):
    h = g_ref[...]
    h = jnp.maximum(_dot(h, w1[...]) + b1[...], 0.0)
    h = jnp.maximum(_dot(h, w2[...]) + b2[...], 0.0)
    h = jnp.maximum(_dot(h, w3[...]) + b3[...], 0.0)
    h_out[...] = _dot(h, w4[...]) + b4[...]


def _edge_mlp(g, ws):
    nblk = EP // RB
    full = pl.BlockSpec((FP, FP), lambda i: (0, 0))
    brow = pl.BlockSpec((1, FP), lambda i: (0, 0))
    blk = pl.BlockSpec((RB, FP), lambda i: (i, 0))
    args = []
    for (w, b) in ws:
        args += [w, b]
    return pl.pallas_call(
        _edge_mlp_body,
        grid=(nblk,),
        in_specs=[blk] + [full, brow] * 4,
        out_specs=blk,
        out_shape=jax.ShapeDtypeStruct((EP, FP), jnp.float32),
    )(g, *args)


def _segment_stage(h, sdst, offs):
    npw = NP // NW
    nch = FP // FCH
    nsl = FCH // 16
    mesh = plsc.VectorSubcoreMesh(core_axis_name="c", subcore_axis_name="s")

    @functools.partial(
        pl.kernel, mesh=mesh,
        out_type=[jax.ShapeDtypeStruct((NP, FP), jnp.float32)] * 4
        + [jax.ShapeDtypeStruct((NP, 16), jnp.float32)],
        scratch_types=[
            pltpu.VMEM((64,), jnp.int32),
            pltpu.VMEM((EBK + 16,), jnp.int32),
            pltpu.VMEM((EBK, FCH), jnp.float32),
            pltpu.VMEM((NP // NW, FCH), jnp.float32),
            pltpu.VMEM((NP // NW, FCH), jnp.float32),
            pltpu.VMEM((NP // NW, 16), jnp.float32),
        ],
    )
    def seg(h_hbm, dst_hbm, offs_hbm, s_hbm, q_hbm, mn_hbm, mx_hbm, c_hbm,
            offs_v, dst_v, hbuf, acc_a, acc_b, cntv):
        wid = _worker_id()
        node0 = wid * npw
        pltpu.sync_copy(offs_hbm, offs_v.at[pl.ds(0, 48)])
        ov = offs_v[pl.ds(wid, 16)]
        lo = ov[0]
        hi = ov[1]
        t0 = lo // EBK
        t1 = (hi + EBK - 1) // EBK

        def zc(i, _):
            cntv[i, pl.ds(0, 16)] = jnp.zeros((16,), jnp.float32)
            return 0
        lax.fori_loop(0, npw, zc, 0)

        def cnt_tile(t, _):
            e0 = t * EBK
            pltpu.sync_copy(dst_hbm.at[pl.ds(e0, EBK)], dst_v.at[pl.ds(0, EBK)])

            def edge(e, __):
                eg = e0 + e

                @pl.when(jnp.logical_and(eg >= lo, eg < hi))
                def _():
                    n = dst_v[pl.ds(e, 16)][0] - node0
                    cntv[n, pl.ds(0, 16)] = cntv[n, pl.ds(0, 16)] + 1.0
                return 0
            lax.fori_loop(0, EBK, edge, 0)
            return 0
        lax.fori_loop(t0, t1, cnt_tile, 0)
        pltpu.sync_copy(cntv, c_hbm.at[pl.ds(node0, npw)])

        def make_chunk(mode):
            a0 = 0.0 if mode == 0 else BIG
            b0 = 0.0 if mode == 0 else -BIG
            oa, ob = (s_hbm, q_hbm) if mode == 0 else (mn_hbm, mx_hbm)

            def chunk(ci, _):
                c0 = ci * FCH

                def zi(i, __):
                    ra = jnp.full((16,), a0, jnp.float32)
                    rb = jnp.full((16,), b0, jnp.float32)
                    for kk in range(nsl):
                        acc_a[i, pl.ds(kk * 16, 16)] = ra
                        acc_b[i, pl.ds(kk * 16, 16)] = rb
                    return 0
                lax.fori_loop(0, npw, zi, 0)

                def tile(t, __):
                    e0 = t * EBK
                    pltpu.sync_copy(dst_hbm.at[pl.ds(e0, EBK)],
                                    dst_v.at[pl.ds(0, EBK)])
                    pltpu.sync_copy(h_hbm.at[pl.ds(e0, EBK), pl.ds(c0, FCH)],
                                    hbuf)

                    def edge(e, ___):
                        eg = e0 + e

                        @pl.when(jnp.logical_and(eg >= lo, eg < hi))
                        def _():
                            n = dst_v[pl.ds(e, 16)][0] - node0
                            for kk in range(nsl):
                                sl = pl.ds(kk * 16, 16)
                                hv = hbuf[e, sl]
                                if mode == 0:
                                    acc_a[n, sl] = acc_a[n, sl] + hv
                                    acc_b[n, sl] = acc_b[n, sl] + hv * hv
                                else:
                                    acc_a[n, sl] = jnp.minimum(acc_a[n, sl], hv)
                                    acc_b[n, sl] = jnp.maximum(acc_b[n, sl], hv)
                        return 0
                    lax.fori_loop(0, EBK, edge, 0)
                    return 0
                lax.fori_loop(t0, t1, tile, 0)
                pltpu.sync_copy(acc_a, oa.at[pl.ds(node0, npw), pl.ds(c0, FCH)])
                pltpu.sync_copy(acc_b, ob.at[pl.ds(node0, npw), pl.ds(c0, FCH)])
                return 0
            return chunk

        lax.fori_loop(0, nch, make_chunk(0), 0)
        lax.fori_loop(0, nch, make_chunk(1), 0)

    return seg(h, sdst, offs)


def _post_body(x1_ref, s_ref, q_ref, mn_ref, mx_ref, cnt_ref, aw_ref,
               wx, wa, b0, w1, b1, w2, b2, w3, b3, w4, b4,
               out_ref, ps_ref, pq_ref):
    i = pl.program_id(0)
    awv = aw_ref[...]
    ex = jnp.exp(awv - jnp.max(awv))
    sm = ex / jnp.sum(ex)
    lane = jax.lax.broadcasted_iota(jnp.int32, (1, 128), 1)
    wk = [jnp.sum(jnp.where(lane == k, sm, 0.0)) for k in range(5)]
    cnt = cnt_ref[...][:, :1]
    pos = cnt > 0.0
    s = jnp.where(pos, s_ref[...], 0.0)
    q = jnp.where(pos, q_ref[...], 0.0)
    mn = jnp.where(pos, mn_ref[...], 0.0)
    mx = jnp.where(pos, mx_ref[...], 0.0)
    r = 1.0 / jnp.maximum(cnt, 1.0)
    mean = s * r
    std = jnp.sqrt(jnp.maximum(q * r - mean * mean, 0.0) + 1e-5)
    agg = wk[0] * s + wk[1] * mean + wk[2] * mn + wk[3] * mx + wk[4] * std
    h = _dot(x1_ref[...], wx[...]) + _dot(agg, wa[...]) + b0[...]
    h = jnp.maximum(h, 0.0)
    h = jnp.maximum(_dot(h, w1[...]) + b1[...], 0.0)
    h = jnp.maximum(_dot(h, w2[...]) + b2[...], 0.0)
    h = jnp.maximum(_dot(h, w3[...]) + b3[...], 0.0)
    h = _dot(h, w4[...]) + b4[...]
    out_ref[...] = h
    rows = jax.lax.broadcasted_iota(jnp.int32, (RB, 1), 0) + i * RB
    m = (rows < N).astype(jnp.float32)
    hm = h * m
    ps = jnp.sum(hm.reshape(RB // 8, 8, FP), axis=0)
    pq = jnp.sum((hm * hm).reshape(RB // 8, 8, FP), axis=0)

    @pl.when(i == 0)
    def _():
        ps_ref[...] = jnp.zeros_like(ps_ref)
        pq_ref[...] = jnp.zeros_like(pq_ref)

    ps_ref[...] += ps
    pq_ref[...] += pq


def _post_stage(x1, s, q, mn, mx, cnt128, awp, ws):
    nblk = NP // RB
    full = pl.BlockSpec((FP, FP), lambda i: (0, 0))
    brow = pl.BlockSpec((1, FP), lambda i: (0, 0))
    brow128 = pl.BlockSpec((1, 128), lambda i: (0, 0))
    blk = pl.BlockSpec((RB, FP), lambda i: (i, 0))
    blk128 = pl.BlockSpec((RB, 128), lambda i: (i, 0))
    acc = pl.BlockSpec((8, FP), lambda i: (0, 0))
    args = []
    for (w, b) in ws[1:]:
        args += [w, b]
    return pl.pallas_call(
        _post_body,
        grid=(nblk,),
        in_specs=[blk, blk, blk, blk, blk, blk128, brow128,
                  full, full, brow] + [full, brow] * 4,
        out_specs=[blk, acc, acc],
        out_shape=[jax.ShapeDtypeStruct((NP, FP), jnp.float32),
                   jax.ShapeDtypeStruct((8, FP), jnp.float32),
                   jax.ShapeDtypeStruct((8, FP), jnp.float32)],
    )(x1, s, q, mn, mx, cnt128, awp, ws[0][0], ws[0][1], ws[0][2], *args)


def _bn_force_body(out_ref, ps_ref, pq_ref, gam, bet, oh_ref,
                   w1, b1, w2, b2, w3, b3, force_ref, pool_ref):
    i = pl.program_id(0)
    mu = jnp.sum(ps_ref[...], axis=0, keepdims=True) / N
    var = jnp.sum(pq_ref[...], axis=0, keepdims=True) / N - mu * mu
    h = (out_ref[...] - mu) * jax.lax.rsqrt(var + 1e-5) * gam[...] + bet[...]
    h = jnp.maximum(h, 0.0)
    part = jax.lax.dot_general(oh_ref[...], h, (((0,), (0,)), ((), ())),
                               preferred_element_type=jnp.float32)

    @pl.when(i == 0)
    def _():
        pool_ref[...] = jnp.zeros_like(pool_ref)

    pool_ref[...] += part
    f = jnp.maximum(_dot(h, w1[...]) + b1[...], 0.0)
    f = jnp.maximum(_dot(f, w2[...]) + b2[...], 0.0)
    force_ref[...] = _dot(f, w3[...]) + b3[...]


def _bn_force_stage(out, ps, pq, gam, bet, ohp, m3):
    nblk = NP // RB
    blk = pl.BlockSpec((RB, FP), lambda i: (i, 0))
    acc8 = pl.BlockSpec((8, FP), lambda i: (0, 0))
    brow = pl.BlockSpec((1, FP), lambda i: (0, 0))
    bblk = pl.BlockSpec((RB, 128), lambda i: (i, 0))
    poolspec = pl.BlockSpec((128, FP), lambda i: (0, 0))
    (w1, b1), (w2, b2), (w3, b3) = m3
    h1, h2, h3 = w1.shape[1], w2.shape[1], w3.shape[1]
    specs = [blk, acc8, acc8, brow, brow, bblk,
             pl.BlockSpec((FP, h1), lambda i: (0, 0)),
             pl.BlockSpec((1, h1), lambda i: (0, 0)),
             pl.BlockSpec((h1, h2), lambda i: (0, 0)),
             pl.BlockSpec((1, h2), lambda i: (0, 0)),
             pl.BlockSpec((h2, h3), lambda i: (0, 0)),
             pl.BlockSpec((1, h3), lambda i: (0, 0))]
    return pl.pallas_call(
        _bn_force_body,
        grid=(nblk,),
        in_specs=specs,
        out_specs=[pl.BlockSpec((RB, h3), lambda i: (i, 0)), poolspec],
        out_shape=[jax.ShapeDtypeStruct((NP, h3), jnp.float32),
                   jax.ShapeDtypeStruct((128, FP), jnp.float32)],
    )(out, ps, pq, gam, bet, ohp, w1, b1, w2, b2, w3, b3)


def _energy_body(pool_ref, w1, b1, w2, b2, w3, b3, e_ref):
    f = jnp.maximum(_dot(pool_ref[...], w1[...]) + b1[...], 0.0)
    f = jnp.maximum(_dot(f, w2[...]) + b2[...], 0.0)
    e_ref[...] = _dot(f, w3[...]) + b3[...]


def _energy_stage(pool, m2):
    (w1, b1), (w2, b2), (w3, b3) = m2
    h1, h2, h3 = w1.shape[1], w2.shape[1], w3.shape[1]
    full = lambda a: pl.BlockSpec(a.shape, lambda: tuple(0 for _ in a.shape))
    return pl.pallas_call(
        _energy_body,
        in_specs=[full(pool), full(w1), full(b1), full(w2), full(b2),
                  full(w3), full(b3)],
        out_specs=pl.BlockSpec((128, h3), lambda: (0, 0)),
        out_shape=jax.ShapeDtypeStruct((128, h3), jnp.float32),
    )(pool, w1, b1, w2, b2, w3, b3)


def _eemb_body(emb_ref, we_ref, out_ref):
    out_ref[...] = _dot(emb_ref[...], we_ref[...])


def _eemb_stage(embp, wep):
    return pl.pallas_call(
        _eemb_body,
        in_specs=[pl.BlockSpec(embp.shape, lambda: (0, 0)),
                  pl.BlockSpec(wep.shape, lambda: (0, 0))],
        out_specs=pl.BlockSpec((embp.shape[0], FP), lambda: (0, 0)),
        out_shape=jax.ShapeDtypeStruct((embp.shape[0], FP), jnp.float32),
    )(embp, wep)


def kernel(x, edge_index, edge_attr, batch, edge_emb, agg_weights,
           mlp1, pre_nn, post_nn, bn_gamma, bn_beta, mlp2, mlp3):
    xp = _pad2(x, NP, FP)
    w1p = _pad2(mlp1[0][0], FP, FP)
    b1p = _pad1(mlp1[0][1], FP)[None, :]

    w0 = pre_nn[0][0]
    wd = _pad2(w0[:F], FP, FP)
    ws = _pad2(w0[F:2 * F], FP, FP)
    we = w0[2 * F:]
    b0 = _pad1(pre_nn[0][1], FP)[None, :]

    x1, P, Q = _node_stage(xp, w1p, b1p, wd, b0, ws)

    ed = edge_emb.shape[1]
    embp = _pad2(edge_emb, 32, 16)
    wep = _pad2(we, 16, FP)
    Eemb = _eemb_stage(embp, wep)

    src = edge_index[0]
    dst = edge_index[1]
    dstp = jnp.concatenate([dst, jnp.full((EP - E,), NP - 1, jnp.int32)])
    srcp = jnp.concatenate([src, jnp.zeros((EP - E,), jnp.int32)])
    attrp = jnp.concatenate([edge_attr, jnp.zeros((EP - E,), jnp.int32)])
    perm = jnp.argsort(dstp)
    sdst = dstp[perm]
    ssrc = srcp[perm]
    sattr = attrp[perm]
    npw = NP // NW
    offs = jnp.searchsorted(
        sdst, jnp.arange(NW + 1, dtype=jnp.int32) * npw).astype(jnp.int32)
    offsp = jnp.pad(offs, (0, 48 - (NW + 1)))

    g = jnp.maximum(P[sdst] + Q[ssrc] + Eemb[sattr], 0.0)

    pre_ws = [(_pad2(w, FP, FP), _pad1(b, FP)[None, :]) for (w, b) in pre_nn[1:]]
    h = _edge_mlp(g, pre_ws)

    s, q, mn, mx, cnt16 = _segment_stage(h, sdst, offsp)
    cnt128 = jnp.broadcast_to(cnt16[:, :1], (NP, 128))
    awp = jnp.full((1, 128), -1e30, jnp.float32).at[0, :5].set(agg_weights)

    pw0 = post_nn[0][0]
    wx = _pad2(pw0[:F], FP, FP)
    wa = _pad2(pw0[F:], FP, FP)
    pb0 = _pad1(post_nn[0][1], FP)[None, :]
    post_ws = [(wx, wa, pb0)] + [(_pad2(w_, FP, FP), _pad1(b_, FP)[None, :])
                                 for (w_, b_) in post_nn[1:]]
    out, ps, pq = _post_stage(x1, s, q, mn, mx, cnt128, awp, post_ws)

    gam = _pad1(bn_gamma, FP)[None, :]
    bet = _pad1(bn_beta, FP)[None, :]
    batchp = jnp.concatenate([batch, jnp.full((NP - N,), NG, jnp.int32)])
    ohp = (batchp[:, None] == jnp.arange(128)[None, :]).astype(jnp.float32)

    def padmlp(m):
        dims = [FP] + [((w_.shape[1] + 127) // 128) * 128 for (w_, _) in m]
        return [(_pad2(w_, dims[i], dims[i + 1]),
                 _pad1(b_, dims[i + 1])[None, :]) for i, (w_, b_) in enumerate(m)]

    m3 = padmlp(mlp3)
    force_p, pool = _bn_force_stage(out, ps, pq, gam, bet, ohp, m3)
    m2 = padmlp(mlp2)
    energy_p = _energy_stage(pool, m2)

    force = force_p[:N, :3]
    energy = energy_p[:NG, :1]
    return force, energy, jnp.float32(1.0)


# speedup vs baseline: 1.0921x; 1.0921x over previous
import functools
import jax
import jax.numpy as jnp
from jax import lax
from jax.experimental import pallas as pl
from jax.experimental.pallas import tpu as pltpu
from jax.experimental.pallas import tpu_sc as plsc

F = 1262
FP = 1280
N = 10000
NP = 10240
E = 40000
EP = 40960
NG = 16
RB = 256


def _worker_id():
    return lax.axis_index("s") * 2 + lax.axis_index("c")


NW = 32
EBK = 64
FCH = 128
BIG = 3.0e38


def _pad2(a, r, c):
    return jnp.pad(a, ((0, r - a.shape[0]), (0, c - a.shape[1])))


def _pad1(a, n):
    return jnp.pad(a, ((0, n - a.shape[0]),))


def _dot(a, b):
    return jnp.dot(a, b, preferred_element_type=jnp.float32)


def _node_body(x_ref, w1, b1, wd, b0, ws, x1_out, p_out, q_out):
    x1 = jnp.maximum(_dot(x_ref[...], w1[...]) + b1[...], 0.0)
    x1_out[...] = x1
    p_out[...] = _dot(x1, wd[...]) + b0[...]
    q_out[...] = _dot(x1, ws[...])


def _node_stage(xp, w1, b1, wd, b0, ws):
    nblk = NP // RB
    full = pl.BlockSpec((FP, FP), lambda i: (0, 0))
    brow = pl.BlockSpec((1, FP), lambda i: (0, 0))
    blk = pl.BlockSpec((RB, FP), lambda i: (i, 0))
    return pl.pallas_call(
        _node_body,
        grid=(nblk,),
        in_specs=[blk, full, brow, full, brow, full],
        out_specs=[blk, blk, blk],
        out_shape=[jax.ShapeDtypeStruct((NP, FP), jnp.float32)] * 3,
    )(xp, w1, b1, wd, b0, ws)


def _edge_mlp_body(g_ref, w1, b1, w2, b2, w3, b3, w4, b4, h_out):
    h = g_ref[...]
    h = jnp.maximum(_dot(h, w1[...]) + b1[...], 0.0)
    h = jnp.maximum(_dot(h, w2[...]) + b2[...], 0.0)
    h = jnp.maximum(_dot(h, w3[...]) + b3[...], 0.0)
    h_out[...] = _dot(h, w4[...]) + b4[...]


def _edge_mlp(g, ws):
    nblk = EP // RB
    full = pl.BlockSpec((FP, FP), lambda i: (0, 0))
    brow = pl.BlockSpec((1, FP), lambda i: (0, 0))
    blk = pl.BlockSpec((RB, FP), lambda i: (i, 0))
    args = []
    for (w, b) in ws:
        args += [w, b]
    return pl.pallas_call(
        _edge_mlp_body,
        grid=(nblk,),
        in_specs=[blk] + [full, brow] * 4,
        out_specs=blk,
        out_shape=jax.ShapeDtypeStruct((EP, FP), jnp.float32),
    )(g, *args)


def _segment_stage(h, sdst, offs):
    npw = NP // NW
    nch = FP // FCH
    nsl = FCH // 16
    mesh = plsc.VectorSubcoreMesh(core_axis_name="c", subcore_axis_name="s")

    @functools.partial(
        pl.kernel, mesh=mesh,
        out_type=[jax.ShapeDtypeStruct((NP, FP), jnp.float32)] * 4
        + [jax.ShapeDtypeStruct((NP, FCH), jnp.float32)],
        scratch_types=[
            pltpu.VMEM((64,), jnp.int32),
            pltpu.VMEM((EBK + 16,), jnp.int32),
            pltpu.VMEM((EBK, FCH), jnp.float32),
            pltpu.VMEM((NP // NW, FCH), jnp.float32),
            pltpu.VMEM((NP // NW, FCH), jnp.float32),
        ],
    )
    def seg(h_hbm, dst_hbm, offs_hbm, s_hbm, q_hbm, mn_hbm, mx_hbm, c_hbm,
            offs_v, dst_v, hbuf, acc_a, acc_b):
        wid = _worker_id()
        node0 = wid * npw
        pltpu.sync_copy(offs_hbm, offs_v.at[pl.ds(0, 48)])
        ov = offs_v[pl.ds(wid, 16)]
        lo = ov[0]
        hi = ov[1]
        t0 = lo // EBK
        t1 = (hi + EBK - 1) // EBK

        def zc(i, _):
            z = jnp.zeros((16,), jnp.float32)
            for kk in range(nsl):
                acc_b[i, pl.ds(kk * 16, 16)] = z
            return 0
        lax.fori_loop(0, npw, zc, 0, unroll=8)

        def cnt_tile(t, _):
            e0 = t * EBK
            pltpu.sync_copy(dst_hbm.at[pl.ds(e0, EBK)], dst_v.at[pl.ds(0, EBK)])

            def edge(e, __):
                eg = e0 + e

                @pl.when(jnp.logical_and(eg >= lo, eg < hi))
                def _():
                    n = dst_v[pl.ds(e, 16)][0] - node0
                    for kk in range(nsl):
                        sl = pl.ds(kk * 16, 16)
                        acc_b[n, sl] = acc_b[n, sl] + 1.0
                return 0
            lax.fori_loop(0, EBK, edge, 0, unroll=4)
            return 0
        lax.fori_loop(t0, t1, cnt_tile, 0)
        pltpu.sync_copy(acc_b, c_hbm.at[pl.ds(node0, npw)])

        def make_chunk(mode):
            a0 = 0.0 if mode == 0 else BIG
            b0 = 0.0 if mode == 0 else -BIG
            oa, ob = (s_hbm, q_hbm) if mode == 0 else (mn_hbm, mx_hbm)

            def chunk(ci, _):
                c0 = ci * FCH

                def zi(i, __):
                    ra = jnp.full((16,), a0, jnp.float32)
                    rb = jnp.full((16,), b0, jnp.float32)
                    for kk in range(nsl):
                        acc_a[i, pl.ds(kk * 16, 16)] = ra
                        acc_b[i, pl.ds(kk * 16, 16)] = rb
                    return 0
                lax.fori_loop(0, npw, zi, 0, unroll=8)

                def tile(t, __):
                    e0 = t * EBK
                    pltpu.sync_copy(dst_hbm.at[pl.ds(e0, EBK)],
                                    dst_v.at[pl.ds(0, EBK)])
                    pltpu.sync_copy(h_hbm.at[pl.ds(e0, EBK), pl.ds(c0, FCH)],
                                    hbuf)

                    def edge(e, ___):
                        eg = e0 + e

                        @pl.when(jnp.logical_and(eg >= lo, eg < hi))
                        def _():
                            n = dst_v[pl.ds(e, 16)][0] - node0
                            for kk in range(nsl):
                                sl = pl.ds(kk * 16, 16)
                                hv = hbuf[e, sl]
                                if mode == 0:
                                    acc_a[n, sl] = acc_a[n, sl] + hv
                                    acc_b[n, sl] = acc_b[n, sl] + hv * hv
                                else:
                                    acc_a[n, sl] = jnp.minimum(acc_a[n, sl], hv)
                                    acc_b[n, sl] = jnp.maximum(acc_b[n, sl], hv)
                        return 0
                    lax.fori_loop(0, EBK, edge, 0, unroll=2)
                    return 0
                lax.fori_loop(t0, t1, tile, 0)
                pltpu.sync_copy(acc_a, oa.at[pl.ds(node0, npw), pl.ds(c0, FCH)])
                pltpu.sync_copy(acc_b, ob.at[pl.ds(node0, npw), pl.ds(c0, FCH)])
                return 0
            return chunk

        lax.fori_loop(0, nch, make_chunk(0), 0)
        lax.fori_loop(0, nch, make_chunk(1), 0)

    return seg(h, sdst, offs)


def _post_body(x1_ref, s_ref, q_ref, mn_ref, mx_ref, cnt_ref, aw_ref,
               wx, wa, b0, w1, b1, w2, b2, w3, b3, w4, b4,
               out_ref, ps_ref, pq_ref):
    i = pl.program_id(0)
    awv = aw_ref[...]
    ex = jnp.exp(awv - jnp.max(awv))
    sm = ex / jnp.sum(ex)
    lane = jax.lax.broadcasted_iota(jnp.int32, (1, 128), 1)
    wk = [jnp.sum(jnp.where(lane == k, sm, 0.0)) for k in range(5)]
    cnt = cnt_ref[...][:, :1]
    pos = cnt > 0.0
    s = jnp.where(pos, s_ref[...], 0.0)
    q = jnp.where(pos, q_ref[...], 0.0)
    mn = jnp.where(pos, mn_ref[...], 0.0)
    mx = jnp.where(pos, mx_ref[...], 0.0)
    r = 1.0 / jnp.maximum(cnt, 1.0)
    mean = s * r
    std = jnp.sqrt(jnp.maximum(q * r - mean * mean, 0.0) + 1e-5)
    agg = wk[0] * s + wk[1] * mean + wk[2] * mn + wk[3] * mx + wk[4] * std
    h = _dot(x1_ref[...], wx[...]) + _dot(agg, wa[...]) + b0[...]
    h = jnp.maximum(h, 0.0)
    h = jnp.maximum(_dot(h, w1[...]) + b1[...], 0.0)
    h = jnp.maximum(_dot(h, w2[...]) + b2[...], 0.0)
    h = jnp.maximum(_dot(h, w3[...]) + b3[...], 0.0)
    h = _dot(h, w4[...]) + b4[...]
    out_ref[...] = h
    rows = jax.lax.broadcasted_iota(jnp.int32, (RB, 1), 0) + i * RB
    m = (rows < N).astype(jnp.float32)
    hm = h * m
    ps = jnp.sum(hm.reshape(RB // 8, 8, FP), axis=0)
    pq = jnp.sum((hm * hm).reshape(RB // 8, 8, FP), axis=0)

    @pl.when(i == 0)
    def _():
        ps_ref[...] = jnp.zeros_like(ps_ref)
        pq_ref[...] = jnp.zeros_like(pq_ref)

    ps_ref[...] += ps
    pq_ref[...] += pq


def _post_stage(x1, s, q, mn, mx, cnt128, awp, ws):
    nblk = NP // RB
    full = pl.BlockSpec((FP, FP), lambda i: (0, 0))
    brow = pl.BlockSpec((1, FP), lambda i: (0, 0))
    brow128 = pl.BlockSpec((1, 128), lambda i: (0, 0))
    blk = pl.BlockSpec((RB, FP), lambda i: (i, 0))
    blk128 = pl.BlockSpec((RB, FCH), lambda i: (i, 0))
    acc = pl.BlockSpec((8, FP), lambda i: (0, 0))
    args = []
    for (w, b) in ws[1:]:
        args += [w, b]
    return pl.pallas_call(
        _post_body,
        grid=(nblk,),
        in_specs=[blk, blk, blk, blk, blk, blk128, brow128,
                  full, full, brow] + [full, brow] * 4,
        out_specs=[blk, acc, acc],
        out_shape=[jax.ShapeDtypeStruct((NP, FP), jnp.float32),
                   jax.ShapeDtypeStruct((8, FP), jnp.float32),
                   jax.ShapeDtypeStruct((8, FP), jnp.float32)],
    )(x1, s, q, mn, mx, cnt128, awp, ws[0][0], ws[0][1], ws[0][2], *args)


def _bn_force_body(out_ref, ps_ref, pq_ref, gam, bet, oh_ref,
                   w1, b1, w2, b2, w3, b3, force_ref, pool_ref):
    i = pl.program_id(0)
    mu = jnp.sum(ps_ref[...], axis=0, keepdims=True) / N
    var = jnp.sum(pq_ref[...], axis=0, keepdims=True) / N - mu * mu
    h = (out_ref[...] - mu) * jax.lax.rsqrt(var + 1e-5) * gam[...] + bet[...]
    h = jnp.maximum(h, 0.0)
    part = jax.lax.dot_general(oh_ref[...], h, (((0,), (0,)), ((), ())),
                               preferred_element_type=jnp.float32)

    @pl.when(i == 0)
    def _():
        pool_ref[...] = jnp.zeros_like(pool_ref)

    pool_ref[...] += part
    f = jnp.maximum(_dot(h, w1[...]) + b1[...], 0.0)
    f = jnp.maximum(_dot(f, w2[...]) + b2[...], 0.0)
    force_ref[...] = _dot(f, w3[...]) + b3[...]


def _bn_force_stage(out, ps, pq, gam, bet, ohp, m3):
    nblk = NP // RB
    blk = pl.BlockSpec((RB, FP), lambda i: (i, 0))
    acc8 = pl.BlockSpec((8, FP), lambda i: (0, 0))
    brow = pl.BlockSpec((1, FP), lambda i: (0, 0))
    bblk = pl.BlockSpec((RB, 128), lambda i: (i, 0))
    poolspec = pl.BlockSpec((128, FP), lambda i: (0, 0))
    (w1, b1), (w2, b2), (w3, b3) = m3
    h1, h2, h3 = w1.shape[1], w2.shape[1], w3.shape[1]
    specs = [blk, acc8, acc8, brow, brow, bblk,
             pl.BlockSpec((FP, h1), lambda i: (0, 0)),
             pl.BlockSpec((1, h1), lambda i: (0, 0)),
             pl.BlockSpec((h1, h2), lambda i: (0, 0)),
             pl.BlockSpec((1, h2), lambda i: (0, 0)),
             pl.BlockSpec((h2, h3), lambda i: (0, 0)),
             pl.BlockSpec((1, h3), lambda i: (0, 0))]
    return pl.pallas_call(
        _bn_force_body,
        grid=(nblk,),
        in_specs=specs,
        out_specs=[pl.BlockSpec((RB, h3), lambda i: (i, 0)), poolspec],
        out_shape=[jax.ShapeDtypeStruct((NP, h3), jnp.float32),
                   jax.ShapeDtypeStruct((128, FP), jnp.float32)],
    )(out, ps, pq, gam, bet, ohp, w1, b1, w2, b2, w3, b3)


def _energy_body(pool_ref, w1, b1, w2, b2, w3, b3, e_ref):
    f = jnp.maximum(_dot(pool_ref[...], w1[...]) + b1[...], 0.0)
    f = jnp.maximum(_dot(f, w2[...]) + b2[...], 0.0)
    e_ref[...] = _dot(f, w3[...]) + b3[...]


def _energy_stage(pool, m2):
    (w1, b1), (w2, b2), (w3, b3) = m2
    h1, h2, h3 = w1.shape[1], w2.shape[1], w3.shape[1]
    full = lambda a: pl.BlockSpec(a.shape, lambda: tuple(0 for _ in a.shape))
    return pl.pallas_call(
        _energy_body,
        in_specs=[full(pool), full(w1), full(b1), full(w2), full(b2),
                  full(w3), full(b3)],
        out_specs=pl.BlockSpec((128, h3), lambda: (0, 0)),
        out_shape=jax.ShapeDtypeStruct((128, h3), jnp.float32),
    )(pool, w1, b1, w2, b2, w3, b3)


def _eemb_body(emb_ref, we_ref, out_ref):
    out_ref[...] = _dot(emb_ref[...], we_ref[...])


def _eemb_stage(embp, wep):
    return pl.pallas_call(
        _eemb_body,
        in_specs=[pl.BlockSpec(embp.shape, lambda: (0, 0)),
                  pl.BlockSpec(wep.shape, lambda: (0, 0))],
        out_specs=pl.BlockSpec((embp.shape[0], FP), lambda: (0, 0)),
        out_shape=jax.ShapeDtypeStruct((embp.shape[0], FP), jnp.float32),
    )(embp, wep)


def kernel(x, edge_index, edge_attr, batch, edge_emb, agg_weights,
           mlp1, pre_nn, post_nn, bn_gamma, bn_beta, mlp2, mlp3):
    xp = _pad2(x, NP, FP)
    w1p = _pad2(mlp1[0][0], FP, FP)
    b1p = _pad1(mlp1[0][1], FP)[None, :]

    w0 = pre_nn[0][0]
    wd = _pad2(w0[:F], FP, FP)
    ws = _pad2(w0[F:2 * F], FP, FP)
    we = w0[2 * F:]
    b0 = _pad1(pre_nn[0][1], FP)[None, :]

    x1, P, Q = _node_stage(xp, w1p, b1p, wd, b0, ws)

    ed = edge_emb.shape[1]
    embp = _pad2(edge_emb, 32, 16)
    wep = _pad2(we, 16, FP)
    Eemb = _eemb_stage(embp, wep)

    src = edge_index[0]
    dst = edge_index[1]
    dstp = jnp.concatenate([dst, jnp.full((EP - E,), NP - 1, jnp.int32)])
    srcp = jnp.concatenate([src, jnp.zeros((EP - E,), jnp.int32)])
    attrp = jnp.concatenate([edge_attr, jnp.zeros((EP - E,), jnp.int32)])
    perm = jnp.argsort(dstp)
    sdst = dstp[perm]
    ssrc = srcp[perm]
    sattr = attrp[perm]
    npw = NP // NW
    offs = jnp.searchsorted(
        sdst, jnp.arange(NW + 1, dtype=jnp.int32) * npw).astype(jnp.int32)
    offsp = jnp.pad(offs, (0, 48 - (NW + 1)))

    g = jnp.maximum(P[sdst] + Q[ssrc] + Eemb[sattr], 0.0)

    pre_ws = [(_pad2(w, FP, FP), _pad1(b, FP)[None, :]) for (w, b) in pre_nn[1:]]
    h = _edge_mlp(g, pre_ws)

    s, q, mn, mx, cnt128 = _segment_stage(h, sdst, offsp)
    awp = jnp.full((1, 128), -1e30, jnp.float32).at[0, :5].set(agg_weights)

    pw0 = post_nn[0][0]
    wx = _pad2(pw0[:F], FP, FP)
    wa = _pad2(pw0[F:], FP, FP)
    pb0 = _pad1(post_nn[0][1], FP)[None, :]
    post_ws = [(wx, wa, pb0)] + [(_pad2(w_, FP, FP), _pad1(b_, FP)[None, :])
                                 for (w_, b_) in post_nn[1:]]
    out, ps, pq = _post_stage(x1, s, q, mn, mx, cnt128, awp, post_ws)

    gam = _pad1(bn_gamma, FP)[None, :]
    bet = _pad1(bn_beta, FP)[None, :]
    batchp = jnp.concatenate([batch, jnp.full((NP - N,), NG, jnp.int32)])
    ohp = (batchp[:, None] == jnp.arange(128)[None, :]).astype(jnp.float32)

    def padmlp(m):
        dims = [FP] + [((w_.shape[1] + 127) // 128) * 128 for (w_, _) in m]
        return [(_pad2(w_, dims[i], dims[i + 1]),
                 _pad1(b_, dims[i + 1])[None, :]) for i, (w_, b_) in enumerate(m)]

    m3 = padmlp(mlp3)
    force_p, pool = _bn_force_stage(out, ps, pq, gam, bet, ohp, m3)
    m2 = padmlp(mlp2)
    energy_p = _energy_stage(pool, m2)

    force = force_p[:N, :3]
    energy = energy_p[:NG, :1]
    return force, energy, jnp.float32(1.0)


# speedup vs baseline: 1.1096x; 1.0161x over previous
import functools
import jax
import jax.numpy as jnp
from jax import lax
from jax.experimental import pallas as pl
from jax.experimental.pallas import tpu as pltpu
from jax.experimental.pallas import tpu_sc as plsc

F = 1262
FP = 1280
N = 10000
NP = 10240
E = 40000
EP = 40960
NG = 16
RB = 256


def _worker_id():
    return lax.axis_index("s") * 2 + lax.axis_index("c")


NW = 32
EBK = 64
FCH = 128
BIG = 3.0e38


def _pad2(a, r, c):
    return jnp.pad(a, ((0, r - a.shape[0]), (0, c - a.shape[1])))


def _pad1(a, n):
    return jnp.pad(a, ((0, n - a.shape[0]),))


def _dot(a, b):
    return jnp.dot(a, b, preferred_element_type=jnp.float32)


def _node_body(x_ref, w1, b1, wd, b0, ws, x1_out, p_out, q_out):
    x1 = jnp.maximum(_dot(x_ref[...], w1[...]) + b1[...], 0.0)
    x1_out[...] = x1
    p_out[...] = _dot(x1, wd[...]) + b0[...]
    q_out[...] = _dot(x1, ws[...])


def _node_stage(xp, w1, b1, wd, b0, ws):
    nblk = NP // RB
    full = pl.BlockSpec((FP, FP), lambda i: (0, 0))
    brow = pl.BlockSpec((1, FP), lambda i: (0, 0))
    blk = pl.BlockSpec((RB, FP), lambda i: (i, 0))
    return pl.pallas_call(
        _node_body,
        grid=(nblk,),
        in_specs=[blk, full, brow, full, brow, full],
        out_specs=[blk, blk, blk],
        out_shape=[jax.ShapeDtypeStruct((NP, FP), jnp.float32)] * 3,
    )(xp, w1, b1, wd, b0, ws)


GB = 32


def _gather_stage(P, Q, sdst, ssrc):
    epw = EP // NW
    ntl = epw // GB
    mesh = plsc.VectorSubcoreMesh(core_axis_name="c", subcore_axis_name="s")

    @functools.partial(
        pl.kernel, mesh=mesh,
        out_type=[jax.ShapeDtypeStruct((EP, FP), jnp.float32)] * 2,
        scratch_types=[
            pltpu.VMEM((GB,), jnp.int32),
            pltpu.VMEM((GB, FP), jnp.float32),
            pltpu.SemaphoreType.DMA,
        ],
    )
    def gk(p_hbm, q_hbm, d_hbm, s_hbm, a_hbm, b_hbm, idx_v, buf, sem):
        wid = _worker_id()
        base = wid * epw

        def tile(t, _):
            e0 = base + t * GB
            pltpu.sync_copy(d_hbm.at[pl.ds(e0, GB)], idx_v)
            pltpu.async_copy(p_hbm.at[idx_v], buf, sem).wait()
            pltpu.sync_copy(buf, a_hbm.at[pl.ds(e0, GB)])
            pltpu.sync_copy(s_hbm.at[pl.ds(e0, GB)], idx_v)
            pltpu.async_copy(q_hbm.at[idx_v], buf, sem).wait()
            pltpu.sync_copy(buf, b_hbm.at[pl.ds(e0, GB)])
            return 0
        lax.fori_loop(0, ntl, tile, 0)

    return gk(P, Q, sdst, ssrc)


def _edge_mlp_body(a_ref, b_ref, oh_ref, eemb, w1, b1, w2, b2, w3, b3,
                   w4, b4, h_out):
    e = jnp.dot(oh_ref[...], eemb[...], preferred_element_type=jnp.float32,
                precision=jax.lax.Precision.HIGHEST)
    h = jnp.maximum(a_ref[...] + b_ref[...] + e, 0.0)
    h = jnp.maximum(_dot(h, w1[...]) + b1[...], 0.0)
    h = jnp.maximum(_dot(h, w2[...]) + b2[...], 0.0)
    h = jnp.maximum(_dot(h, w3[...]) + b3[...], 0.0)
    h_out[...] = _dot(h, w4[...]) + b4[...]


def _edge_mlp(a, b, oh, eemb, ws):
    nblk = EP // RB
    full = pl.BlockSpec((FP, FP), lambda i: (0, 0))
    brow = pl.BlockSpec((1, FP), lambda i: (0, 0))
    blk = pl.BlockSpec((RB, FP), lambda i: (i, 0))
    ohspec = pl.BlockSpec((RB, 128), lambda i: (i, 0))
    embspec = pl.BlockSpec((128, FP), lambda i: (0, 0))
    args = []
    for (w, b_) in ws:
        args += [w, b_]
    return pl.pallas_call(
        _edge_mlp_body,
        grid=(nblk,),
        in_specs=[blk, blk, ohspec, embspec] + [full, brow] * 4,
        out_specs=blk,
        out_shape=jax.ShapeDtypeStruct((EP, FP), jnp.float32),
    )(a, b, oh, eemb, *args)


def _segment_stage(h, sdst, offs):
    npw = NP // NW
    nch = FP // FCH
    nsl = FCH // 16
    mesh = plsc.VectorSubcoreMesh(core_axis_name="c", subcore_axis_name="s")

    @functools.partial(
        pl.kernel, mesh=mesh,
        out_type=[jax.ShapeDtypeStruct((NP, FP), jnp.float32)] * 4
        + [jax.ShapeDtypeStruct((NP, FCH), jnp.float32)],
        scratch_types=[
            pltpu.VMEM((64,), jnp.int32),
            pltpu.VMEM((EBK + 16,), jnp.int32),
            pltpu.VMEM((EBK, FCH), jnp.float32),
            pltpu.VMEM((NP // NW, FCH), jnp.float32),
            pltpu.VMEM((NP // NW, FCH), jnp.float32),
        ],
    )
    def seg(h_hbm, dst_hbm, offs_hbm, s_hbm, q_hbm, mn_hbm, mx_hbm, c_hbm,
            offs_v, dst_v, hbuf, acc_a, acc_b):
        wid = _worker_id()
        node0 = wid * npw
        pltpu.sync_copy(offs_hbm, offs_v.at[pl.ds(0, 48)])
        ov = offs_v[pl.ds(wid, 16)]
        lo = ov[0]
        hi = ov[1]
        t0 = lo // EBK
        t1 = (hi + EBK - 1) // EBK

        def zc(i, _):
            z = jnp.zeros((16,), jnp.float32)
            for kk in range(nsl):
                acc_b[i, pl.ds(kk * 16, 16)] = z
            return 0
        lax.fori_loop(0, npw, zc, 0, unroll=8)

        def cnt_tile(t, _):
            e0 = t * EBK
            pltpu.sync_copy(dst_hbm.at[pl.ds(e0, EBK)], dst_v.at[pl.ds(0, EBK)])

            def edge(e, __):
                eg = e0 + e

                @pl.when(jnp.logical_and(eg >= lo, eg < hi))
                def _():
                    n = dst_v[pl.ds(e, 16)][0] - node0
                    for kk in range(nsl):
                        sl = pl.ds(kk * 16, 16)
                        acc_b[n, sl] = acc_b[n, sl] + 1.0
                return 0
            lax.fori_loop(0, EBK, edge, 0, unroll=4)
            return 0
        lax.fori_loop(t0, t1, cnt_tile, 0)
        pltpu.sync_copy(acc_b, c_hbm.at[pl.ds(node0, npw)])

        def make_chunk(mode):
            a0 = 0.0 if mode == 0 else BIG
            b0 = 0.0 if mode == 0 else -BIG
            oa, ob = (s_hbm, q_hbm) if mode == 0 else (mn_hbm, mx_hbm)

            def chunk(ci, _):
                c0 = ci * FCH

                def zi(i, __):
                    ra = jnp.full((16,), a0, jnp.float32)
                    rb = jnp.full((16,), b0, jnp.float32)
                    for kk in range(nsl):
                        acc_a[i, pl.ds(kk * 16, 16)] = ra
                        acc_b[i, pl.ds(kk * 16, 16)] = rb
                    return 0
                lax.fori_loop(0, npw, zi, 0, unroll=8)

                def tile(t, __):
                    e0 = t * EBK
                    pltpu.sync_copy(dst_hbm.at[pl.ds(e0, EBK)],
                                    dst_v.at[pl.ds(0, EBK)])
                    pltpu.sync_copy(h_hbm.at[pl.ds(e0, EBK), pl.ds(c0, FCH)],
                                    hbuf)

                    def edge(e, ___):
                        eg = e0 + e

                        @pl.when(jnp.logical_and(eg >= lo, eg < hi))
                        def _():
                            n = dst_v[pl.ds(e, 16)][0] - node0
                            for kk in range(nsl):
                                sl = pl.ds(kk * 16, 16)
                                hv = hbuf[e, sl]
                                if mode == 0:
                                    acc_a[n, sl] = acc_a[n, sl] + hv
                                    acc_b[n, sl] = acc_b[n, sl] + hv * hv
                                else:
                                    acc_a[n, sl] = jnp.minimum(acc_a[n, sl], hv)
                                    acc_b[n, sl] = jnp.maximum(acc_b[n, sl], hv)
                        return 0
                    lax.fori_loop(0, EBK, edge, 0, unroll=2)
                    return 0
                lax.fori_loop(t0, t1, tile, 0)
                pltpu.sync_copy(acc_a, oa.at[pl.ds(node0, npw), pl.ds(c0, FCH)])
                pltpu.sync_copy(acc_b, ob.at[pl.ds(node0, npw), pl.ds(c0, FCH)])
                return 0
            return chunk

        lax.fori_loop(0, nch, make_chunk(0), 0)
        lax.fori_loop(0, nch, make_chunk(1), 0)

    return seg(h, sdst, offs)


def _post_body(x1_ref, s_ref, q_ref, mn_ref, mx_ref, cnt_ref, aw_ref,
               wx, wa, b0, w1, b1, w2, b2, w3, b3, w4, b4,
               out_ref, ps_ref, pq_ref):
    i = pl.program_id(0)
    awv = aw_ref[...]
    ex = jnp.exp(awv - jnp.max(awv))
    sm = ex / jnp.sum(ex)
    lane = jax.lax.broadcasted_iota(jnp.int32, (1, 128), 1)
    wk = [jnp.sum(jnp.where(lane == k, sm, 0.0)) for k in range(5)]
    cnt = cnt_ref[...][:, :1]
    pos = cnt > 0.0
    s = jnp.where(pos, s_ref[...], 0.0)
    q = jnp.where(pos, q_ref[...], 0.0)
    mn = jnp.where(pos, mn_ref[...], 0.0)
    mx = jnp.where(pos, mx_ref[...], 0.0)
    r = 1.0 / jnp.maximum(cnt, 1.0)
    mean = s * r
    std = jnp.sqrt(jnp.maximum(q * r - mean * mean, 0.0) + 1e-5)
    agg = wk[0] * s + wk[1] * mean + wk[2] * mn + wk[3] * mx + wk[4] * std
    h = _dot(x1_ref[...], wx[...]) + _dot(agg, wa[...]) + b0[...]
    h = jnp.maximum(h, 0.0)
    h = jnp.maximum(_dot(h, w1[...]) + b1[...], 0.0)
    h = jnp.maximum(_dot(h, w2[...]) + b2[...], 0.0)
    h = jnp.maximum(_dot(h, w3[...]) + b3[...], 0.0)
    h = _dot(h, w4[...]) + b4[...]
    out_ref[...] = h
    rows = jax.lax.broadcasted_iota(jnp.int32, (RB, 1), 0) + i * RB
    m = (rows < N).astype(jnp.float32)
    hm = h * m
    ps = jnp.sum(hm.reshape(RB // 8, 8, FP), axis=0)
    pq = jnp.sum((hm * hm).reshape(RB // 8, 8, FP), axis=0)

    @pl.when(i == 0)
    def _():
        ps_ref[...] = jnp.zeros_like(ps_ref)
        pq_ref[...] = jnp.zeros_like(pq_ref)

    ps_ref[...] += ps
    pq_ref[...] += pq


def _post_stage(x1, s, q, mn, mx, cnt128, awp, ws):
    nblk = NP // RB
    full = pl.BlockSpec((FP, FP), lambda i: (0, 0))
    brow = pl.BlockSpec((1, FP), lambda i: (0, 0))
    brow128 = pl.BlockSpec((1, 128), lambda i: (0, 0))
    blk = pl.BlockSpec((RB, FP), lambda i: (i, 0))
    blk128 = pl.BlockSpec((RB, FCH), lambda i: (i, 0))
    acc = pl.BlockSpec((8, FP), lambda i: (0, 0))
    args = []
    for (w, b) in ws[1:]:
        args += [w, b]
    return pl.pallas_call(
        _post_body,
        grid=(nblk,),
        in_specs=[blk, blk, blk, blk, blk, blk128, brow128,
                  full, full, brow] + [full, brow] * 4,
        out_specs=[blk, acc, acc],
        out_shape=[jax.ShapeDtypeStruct((NP, FP), jnp.float32),
                   jax.ShapeDtypeStruct((8, FP), jnp.float32),
                   jax.ShapeDtypeStruct((8, FP), jnp.float32)],
    )(x1, s, q, mn, mx, cnt128, awp, ws[0][0], ws[0][1], ws[0][2], *args)


def _bn_force_body(out_ref, ps_ref, pq_ref, gam, bet, oh_ref,
                   w1, b1, w2, b2, w3, b3, force_ref, pool_ref):
    i = pl.program_id(0)
    mu = jnp.sum(ps_ref[...], axis=0, keepdims=True) / N
    var = jnp.sum(pq_ref[...], axis=0, keepdims=True) / N - mu * mu
    h = (out_ref[...] - mu) * jax.lax.rsqrt(var + 1e-5) * gam[...] + bet[...]
    h = jnp.maximum(h, 0.0)
    part = jax.lax.dot_general(oh_ref[...], h, (((0,), (0,)), ((), ())),
                               preferred_element_type=jnp.float32)

    @pl.when(i == 0)
    def _():
        pool_ref[...] = jnp.zeros_like(pool_ref)

    pool_ref[...] += part
    f = jnp.maximum(_dot(h, w1[...]) + b1[...], 0.0)
    f = jnp.maximum(_dot(f, w2[...]) + b2[...], 0.0)
    force_ref[...] = _dot(f, w3[...]) + b3[...]


def _bn_force_stage(out, ps, pq, gam, bet, ohp, m3):
    nblk = NP // RB
    blk = pl.BlockSpec((RB, FP), lambda i: (i, 0))
    acc8 = pl.BlockSpec((8, FP), lambda i: (0, 0))
    brow = pl.BlockSpec((1, FP), lambda i: (0, 0))
    bblk = pl.BlockSpec((RB, 128), lambda i: (i, 0))
    poolspec = pl.BlockSpec((128, FP), lambda i: (0, 0))
    (w1, b1), (w2, b2), (w3, b3) = m3
    h1, h2, h3 = w1.shape[1], w2.shape[1], w3.shape[1]
    specs = [blk, acc8, acc8, brow, brow, bblk,
             pl.BlockSpec((FP, h1), lambda i: (0, 0)),
             pl.BlockSpec((1, h1), lambda i: (0, 0)),
             pl.BlockSpec((h1, h2), lambda i: (0, 0)),
             pl.BlockSpec((1, h2), lambda i: (0, 0)),
             pl.BlockSpec((h2, h3), lambda i: (0, 0)),
             pl.BlockSpec((1, h3), lambda i: (0, 0))]
    return pl.pallas_call(
        _bn_force_body,
        grid=(nblk,),
        in_specs=specs,
        out_specs=[pl.BlockSpec((RB, h3), lambda i: (i, 0)), poolspec],
        out_shape=[jax.ShapeDtypeStruct((NP, h3), jnp.float32),
                   jax.ShapeDtypeStruct((128, FP), jnp.float32)],
    )(out, ps, pq, gam, bet, ohp, w1, b1, w2, b2, w3, b3)


def _energy_body(pool_ref, w1, b1, w2, b2, w3, b3, e_ref):
    f = jnp.maximum(_dot(pool_ref[...], w1[...]) + b1[...], 0.0)
    f = jnp.maximum(_dot(f, w2[...]) + b2[...], 0.0)
    e_ref[...] = _dot(f, w3[...]) + b3[...]


def _energy_stage(pool, m2):
    (w1, b1), (w2, b2), (w3, b3) = m2
    h1, h2, h3 = w1.shape[1], w2.shape[1], w3.shape[1]
    full = lambda a: pl.BlockSpec(a.shape, lambda: tuple(0 for _ in a.shape))
    return pl.pallas_call(
        _energy_body,
        in_specs=[full(pool), full(w1), full(b1), full(w2), full(b2),
                  full(w3), full(b3)],
        out_specs=pl.BlockSpec((128, h3), lambda: (0, 0)),
        out_shape=jax.ShapeDtypeStruct((128, h3), jnp.float32),
    )(pool, w1, b1, w2, b2, w3, b3)


def _eemb_body(emb_ref, we_ref, out_ref):
    out_ref[...] = _dot(emb_ref[...], we_ref[...])


def _eemb_stage(embp, wep):
    return pl.pallas_call(
        _eemb_body,
        in_specs=[pl.BlockSpec(embp.shape, lambda: (0, 0)),
                  pl.BlockSpec(wep.shape, lambda: (0, 0))],
        out_specs=pl.BlockSpec((embp.shape[0], FP), lambda: (0, 0)),
        out_shape=jax.ShapeDtypeStruct((embp.shape[0], FP), jnp.float32),
    )(embp, wep)


def kernel(x, edge_index, edge_attr, batch, edge_emb, agg_weights,
           mlp1, pre_nn, post_nn, bn_gamma, bn_beta, mlp2, mlp3):
    xp = _pad2(x, NP, FP)
    w1p = _pad2(mlp1[0][0], FP, FP)
    b1p = _pad1(mlp1[0][1], FP)[None, :]

    w0 = pre_nn[0][0]
    wd = _pad2(w0[:F], FP, FP)
    ws = _pad2(w0[F:2 * F], FP, FP)
    we = w0[2 * F:]
    b0 = _pad1(pre_nn[0][1], FP)[None, :]

    x1, P, Q = _node_stage(xp, w1p, b1p, wd, b0, ws)

    embp = _pad2(edge_emb, 128, 16)
    wep = _pad2(we, 16, FP)
    Eemb = _eemb_stage(embp, wep)

    src = edge_index[0]
    dst = edge_index[1]
    dstp = jnp.concatenate([dst, jnp.full((EP - E,), NP - 1, jnp.int32)])
    srcp = jnp.concatenate([src, jnp.zeros((EP - E,), jnp.int32)])
    attrp = jnp.concatenate([edge_attr, jnp.zeros((EP - E,), jnp.int32)])
    perm = jnp.argsort(dstp)
    sdst = dstp[perm]
    ssrc = srcp[perm]
    sattr = attrp[perm]
    npw = NP // NW
    offs = jnp.searchsorted(
        sdst, jnp.arange(NW + 1, dtype=jnp.int32) * npw).astype(jnp.int32)
    offsp = jnp.pad(offs, (0, 48 - (NW + 1)))

    A, B = _gather_stage(P, Q, sdst, ssrc)
    ohattr = (sattr[:, None] == jnp.arange(128)[None, :]).astype(jnp.float32)

    pre_ws = [(_pad2(w, FP, FP), _pad1(b, FP)[None, :]) for (w, b) in pre_nn[1:]]
    h = _edge_mlp(A, B, ohattr, Eemb, pre_ws)

    s, q, mn, mx, cnt128 = _segment_stage(h, sdst, offsp)
    awp = jnp.full((1, 128), -1e30, jnp.float32).at[0, :5].set(agg_weights)

    pw0 = post_nn[0][0]
    wx = _pad2(pw0[:F], FP, FP)
    wa = _pad2(pw0[F:], FP, FP)
    pb0 = _pad1(post_nn[0][1], FP)[None, :]
    post_ws = [(wx, wa, pb0)] + [(_pad2(w_, FP, FP), _pad1(b_, FP)[None, :])
                                 for (w_, b_) in post_nn[1:]]
    out, ps, pq = _post_stage(x1, s, q, mn, mx, cnt128, awp, post_ws)

    gam = _pad1(bn_gamma, FP)[None, :]
    bet = _pad1(bn_beta, FP)[None, :]
    batchp = jnp.concatenate([batch, jnp.full((NP - N,), NG, jnp.int32)])
    ohp = (batchp[:, None] == jnp.arange(128)[None, :]).astype(jnp.float32)

    def padmlp(m):
        dims = [FP] + [((w_.shape[1] + 127) // 128) * 128 for (w_, _) in m]
        return [(_pad2(w_, dims[i], dims[i + 1]),
                 _pad1(b_, dims[i + 1])[None, :]) for i, (w_, b_) in enumerate(m)]

    m3 = padmlp(mlp3)
    force_p, pool = _bn_force_stage(out, ps, pq, gam, bet, ohp, m3)
    m2 = padmlp(mlp2)
    energy_p = _energy_stage(pool, m2)

    force = force_p[:N, :3]
    energy = energy_p[:NG, :1]
    return force, energy, jnp.float32(1.0)


# speedup vs baseline: 1.3823x; 1.2457x over previous
import functools
import jax
import jax.numpy as jnp
from jax import lax
from jax.experimental import pallas as pl
from jax.experimental.pallas import tpu as pltpu
from jax.experimental.pallas import tpu_sc as plsc

F = 1262
FP = 1280
N = 10000
NP = 10240
E = 40000
EP = 40960
NG = 16
RB = 256


def _worker_id():
    return lax.axis_index("s") * 2 + lax.axis_index("c")


NW = 32
EBK = 64
FCH = 128
BIG = 3.0e38


def _pad2(a, r, c):
    return jnp.pad(a, ((0, r - a.shape[0]), (0, c - a.shape[1])))


def _pad1(a, n):
    return jnp.pad(a, ((0, n - a.shape[0]),))


def _dot(a, b):
    return jnp.dot(a, b, preferred_element_type=jnp.float32)


def _node_body(x_ref, w1, b1, wd, b0, ws, x1_out, p_out, q_out):
    x1 = jnp.maximum(_dot(x_ref[...], w1[...]) + b1[...], 0.0)
    x1_out[...] = x1
    p_out[...] = _dot(x1, wd[...]) + b0[...]
    q_out[...] = _dot(x1, ws[...])


def _node_stage(xp, w1, b1, wd, b0, ws):
    nblk = NP // RB
    full = pl.BlockSpec((FP, FP), lambda i: (0, 0))
    brow = pl.BlockSpec((1, FP), lambda i: (0, 0))
    blk = pl.BlockSpec((RB, FP), lambda i: (i, 0))
    return pl.pallas_call(
        _node_body,
        grid=(nblk,),
        in_specs=[blk, full, brow, full, brow, full],
        out_specs=[blk, blk, blk],
        out_shape=[jax.ShapeDtypeStruct((NP, FP), jnp.float32)] * 3,
    )(xp, w1, b1, wd, b0, ws)


GB = 32


def _gather_stage(P, Q, sdst, ssrc):
    epw = EP // NW
    ntl = epw // GB
    mesh = plsc.VectorSubcoreMesh(core_axis_name="c", subcore_axis_name="s")

    @functools.partial(
        pl.kernel, mesh=mesh,
        out_type=[jax.ShapeDtypeStruct((EP, FP), jnp.float32)] * 2,
        scratch_types=[
            pltpu.VMEM((GB,), jnp.int32),
            pltpu.VMEM((GB, FP), jnp.float32),
            pltpu.SemaphoreType.DMA,
        ],
    )
    def gk(p_hbm, q_hbm, d_hbm, s_hbm, a_hbm, b_hbm, idx_v, buf, sem):
        wid = _worker_id()
        base = wid * epw

        def tile(t, _):
            e0 = base + t * GB
            pltpu.sync_copy(d_hbm.at[pl.ds(e0, GB)], idx_v)
            pltpu.async_copy(p_hbm.at[idx_v], buf, sem).wait()
            pltpu.sync_copy(buf, a_hbm.at[pl.ds(e0, GB)])
            pltpu.sync_copy(s_hbm.at[pl.ds(e0, GB)], idx_v)
            pltpu.async_copy(q_hbm.at[idx_v], buf, sem).wait()
            pltpu.sync_copy(buf, b_hbm.at[pl.ds(e0, GB)])
            return 0
        lax.fori_loop(0, ntl, tile, 0)

    return gk(P, Q, sdst, ssrc)


def _edge_mlp_body(a_ref, b_ref, oh_ref, eemb, w1, b1, w2, b2, w3, b3,
                   w4, b4, h_out):
    e = jnp.dot(oh_ref[...], eemb[...], preferred_element_type=jnp.float32,
                precision=jax.lax.Precision.HIGHEST)
    h = jnp.maximum(a_ref[...] + b_ref[...] + e, 0.0)
    h = jnp.maximum(_dot(h, w1[...]) + b1[...], 0.0)
    h = jnp.maximum(_dot(h, w2[...]) + b2[...], 0.0)
    h = jnp.maximum(_dot(h, w3[...]) + b3[...], 0.0)
    h_out[...] = _dot(h, w4[...]) + b4[...]


def _edge_mlp(a, b, oh, eemb, ws):
    nblk = EP // RB
    full = pl.BlockSpec((FP, FP), lambda i: (0, 0))
    brow = pl.BlockSpec((1, FP), lambda i: (0, 0))
    blk = pl.BlockSpec((RB, FP), lambda i: (i, 0))
    ohspec = pl.BlockSpec((RB, 128), lambda i: (i, 0))
    embspec = pl.BlockSpec((128, FP), lambda i: (0, 0))
    args = []
    for (w, b_) in ws:
        args += [w, b_]
    return pl.pallas_call(
        _edge_mlp_body,
        grid=(nblk,),
        in_specs=[blk, blk, ohspec, embspec] + [full, brow] * 4,
        out_specs=blk,
        out_shape=jax.ShapeDtypeStruct((EP, FP), jnp.float32),
    )(a, b, oh, eemb, *args)


def _segment_stage(h, sdst, offs):
    npw = NP // NW
    nsb = 2
    nps = npw // nsb
    nch = FP // FCH
    nsl = FCH // 16
    mesh = plsc.VectorSubcoreMesh(core_axis_name="c", subcore_axis_name="s")

    @functools.partial(
        pl.kernel, mesh=mesh,
        out_type=[jax.ShapeDtypeStruct((NP, FP), jnp.float32)] * 4
        + [jax.ShapeDtypeStruct((NP, FCH), jnp.float32)],
        scratch_types=[
            pltpu.VMEM((80,), jnp.int32),
            pltpu.VMEM((EBK + 16,), jnp.int32),
            pltpu.VMEM((EBK, FCH), jnp.float32),
            pltpu.VMEM((NP // NW // 2, FCH), jnp.float32),
            pltpu.VMEM((NP // NW // 2, FCH), jnp.float32),
            pltpu.VMEM((NP // NW // 2, FCH), jnp.float32),
            pltpu.VMEM((NP // NW // 2, FCH), jnp.float32),
        ],
    )
    def seg(h_hbm, dst_hbm, offs_hbm, s_hbm, q_hbm, mn_hbm, mx_hbm, c_hbm,
            offs_v, dst_v, hbuf, acc_s, acc_q, acc_mn, acc_mx):
        wid = _worker_id()
        pltpu.sync_copy(offs_hbm, offs_v.at[pl.ds(0, 72)])
        ov = offs_v[pl.ds(nsb * wid, 16)]

        for sb in range(nsb):
            node0 = wid * npw + sb * nps
            lo = ov[sb]
            hi = ov[sb + 1]
            t0 = lo // EBK
            t1 = (hi + EBK - 1) // EBK

            def zc(i, _):
                z = jnp.zeros((16,), jnp.float32)
                for kk in range(nsl):
                    acc_mx[i, pl.ds(kk * 16, 16)] = z
                return 0
            lax.fori_loop(0, nps, zc, 0, unroll=8)

            def cnt_tile(t, _):
                e0 = t * EBK
                pltpu.sync_copy(dst_hbm.at[pl.ds(e0, EBK)],
                                dst_v.at[pl.ds(0, EBK)])

                def edge(e, __):
                    eg = e0 + e

                    @pl.when(jnp.logical_and(eg >= lo, eg < hi))
                    def _():
                        n = dst_v[pl.ds(e, 16)][0] - node0
                        for kk in range(nsl):
                            sl = pl.ds(kk * 16, 16)
                            acc_mx[n, sl] = acc_mx[n, sl] + 1.0
                    return 0
                lax.fori_loop(0, EBK, edge, 0)
                return 0
            lax.fori_loop(t0, t1, cnt_tile, 0)
            pltpu.sync_copy(acc_mx, c_hbm.at[pl.ds(node0, nps)])

            def chunk(ci, _):
                c0 = ci * FCH

                def zi(i, __):
                    z = jnp.zeros((16,), jnp.float32)
                    rmn = jnp.full((16,), BIG, jnp.float32)
                    rmx = jnp.full((16,), -BIG, jnp.float32)
                    for kk in range(nsl):
                        sl = pl.ds(kk * 16, 16)
                        acc_s[i, sl] = z
                        acc_q[i, sl] = z
                        acc_mn[i, sl] = rmn
                        acc_mx[i, sl] = rmx
                    return 0
                lax.fori_loop(0, nps, zi, 0, unroll=4)

                def tile(t, __):
                    e0 = t * EBK
                    pltpu.sync_copy(dst_hbm.at[pl.ds(e0, EBK)],
                                    dst_v.at[pl.ds(0, EBK)])
                    pltpu.sync_copy(h_hbm.at[pl.ds(e0, EBK), pl.ds(c0, FCH)],
                                    hbuf)

                    def edge(e, ___):
                        eg = e0 + e

                        @pl.when(jnp.logical_and(eg >= lo, eg < hi))
                        def _():
                            n = dst_v[pl.ds(e, 16)][0] - node0
                            for kk in range(nsl):
                                sl = pl.ds(kk * 16, 16)
                                hv = hbuf[e, sl]
                                acc_s[n, sl] = acc_s[n, sl] + hv
                                acc_q[n, sl] = acc_q[n, sl] + hv * hv
                                acc_mn[n, sl] = jnp.minimum(acc_mn[n, sl], hv)
                                acc_mx[n, sl] = jnp.maximum(acc_mx[n, sl], hv)
                        return 0
                    lax.fori_loop(0, EBK, edge, 0)
                    return 0
                lax.fori_loop(t0, t1, tile, 0)
                pltpu.sync_copy(acc_s, s_hbm.at[pl.ds(node0, nps), pl.ds(c0, FCH)])
                pltpu.sync_copy(acc_q, q_hbm.at[pl.ds(node0, nps), pl.ds(c0, FCH)])
                pltpu.sync_copy(acc_mn, mn_hbm.at[pl.ds(node0, nps), pl.ds(c0, FCH)])
                pltpu.sync_copy(acc_mx, mx_hbm.at[pl.ds(node0, nps), pl.ds(c0, FCH)])
                return 0
            lax.fori_loop(0, nch, chunk, 0)

    return seg(h, sdst, offs)


def _post_body(x1_ref, s_ref, q_ref, mn_ref, mx_ref, cnt_ref, aw_ref,
               wx, wa, b0, w1, b1, w2, b2, w3, b3, w4, b4,
               out_ref, ps_ref, pq_ref):
    i = pl.program_id(0)
    awv = aw_ref[...]
    ex = jnp.exp(awv - jnp.max(awv))
    sm = ex / jnp.sum(ex)
    lane = jax.lax.broadcasted_iota(jnp.int32, (1, 128), 1)
    wk = [jnp.sum(jnp.where(lane == k, sm, 0.0)) for k in range(5)]
    cnt = cnt_ref[...][:, :1]
    pos = cnt > 0.0
    s = jnp.where(pos, s_ref[...], 0.0)
    q = jnp.where(pos, q_ref[...], 0.0)
    mn = jnp.where(pos, mn_ref[...], 0.0)
    mx = jnp.where(pos, mx_ref[...], 0.0)
    r = 1.0 / jnp.maximum(cnt, 1.0)
    mean = s * r
    std = jnp.sqrt(jnp.maximum(q * r - mean * mean, 0.0) + 1e-5)
    agg = wk[0] * s + wk[1] * mean + wk[2] * mn + wk[3] * mx + wk[4] * std
    h = _dot(x1_ref[...], wx[...]) + _dot(agg, wa[...]) + b0[...]
    h = jnp.maximum(h, 0.0)
    h = jnp.maximum(_dot(h, w1[...]) + b1[...], 0.0)
    h = jnp.maximum(_dot(h, w2[...]) + b2[...], 0.0)
    h = jnp.maximum(_dot(h, w3[...]) + b3[...], 0.0)
    h = _dot(h, w4[...]) + b4[...]
    out_ref[...] = h
    rows = jax.lax.broadcasted_iota(jnp.int32, (RB, 1), 0) + i * RB
    m = (rows < N).astype(jnp.float32)
    hm = h * m
    ps = jnp.sum(hm.reshape(RB // 8, 8, FP), axis=0)
    pq = jnp.sum((hm * hm).reshape(RB // 8, 8, FP), axis=0)

    @pl.when(i == 0)
    def _():
        ps_ref[...] = jnp.zeros_like(ps_ref)
        pq_ref[...] = jnp.zeros_like(pq_ref)

    ps_ref[...] += ps
    pq_ref[...] += pq


def _post_stage(x1, s, q, mn, mx, cnt128, awp, ws):
    nblk = NP // RB
    full = pl.BlockSpec((FP, FP), lambda i: (0, 0))
    brow = pl.BlockSpec((1, FP), lambda i: (0, 0))
    brow128 = pl.BlockSpec((1, 128), lambda i: (0, 0))
    blk = pl.BlockSpec((RB, FP), lambda i: (i, 0))
    blk128 = pl.BlockSpec((RB, FCH), lambda i: (i, 0))
    acc = pl.BlockSpec((8, FP), lambda i: (0, 0))
    args = []
    for (w, b) in ws[1:]:
        args += [w, b]
    return pl.pallas_call(
        _post_body,
        grid=(nblk,),
        in_specs=[blk, blk, blk, blk, blk, blk128, brow128,
                  full, full, brow] + [full, brow] * 4,
        out_specs=[blk, acc, acc],
        out_shape=[jax.ShapeDtypeStruct((NP, FP), jnp.float32),
                   jax.ShapeDtypeStruct((8, FP), jnp.float32),
                   jax.ShapeDtypeStruct((8, FP), jnp.float32)],
    )(x1, s, q, mn, mx, cnt128, awp, ws[0][0], ws[0][1], ws[0][2], *args)


def _bn_force_body(out_ref, ps_ref, pq_ref, gam, bet, oh_ref,
                   w1, b1, w2, b2, w3, b3, force_ref, pool_ref):
    i = pl.program_id(0)
    mu = jnp.sum(ps_ref[...], axis=0, keepdims=True) / N
    var = jnp.sum(pq_ref[...], axis=0, keepdims=True) / N - mu * mu
    h = (out_ref[...] - mu) * jax.lax.rsqrt(var + 1e-5) * gam[...] + bet[...]
    h = jnp.maximum(h, 0.0)
    part = jax.lax.dot_general(oh_ref[...], h, (((0,), (0,)), ((), ())),
                               preferred_element_type=jnp.float32)

    @pl.when(i == 0)
    def _():
        pool_ref[...] = jnp.zeros_like(pool_ref)

    pool_ref[...] += part
    f = jnp.maximum(_dot(h, w1[...]) + b1[...], 0.0)
    f = jnp.maximum(_dot(f, w2[...]) + b2[...], 0.0)
    force_ref[...] = _dot(f, w3[...]) + b3[...]


def _bn_force_stage(out, ps, pq, gam, bet, ohp, m3):
    nblk = NP // RB
    blk = pl.BlockSpec((RB, FP), lambda i: (i, 0))
    acc8 = pl.BlockSpec((8, FP), lambda i: (0, 0))
    brow = pl.BlockSpec((1, FP), lambda i: (0, 0))
    bblk = pl.BlockSpec((RB, 128), lambda i: (i, 0))
    poolspec = pl.BlockSpec((128, FP), lambda i: (0, 0))
    (w1, b1), (w2, b2), (w3, b3) = m3
    h1, h2, h3 = w1.shape[1], w2.shape[1], w3.shape[1]
    specs = [blk, acc8, acc8, brow, brow, bblk,
             pl.BlockSpec((FP, h1), lambda i: (0, 0)),
             pl.BlockSpec((1, h1), lambda i: (0, 0)),
             pl.BlockSpec((h1, h2), lambda i: (0, 0)),
             pl.BlockSpec((1, h2), lambda i: (0, 0)),
             pl.BlockSpec((h2, h3), lambda i: (0, 0)),
             pl.BlockSpec((1, h3), lambda i: (0, 0))]
    return pl.pallas_call(
        _bn_force_body,
        grid=(nblk,),
        in_specs=specs,
        out_specs=[pl.BlockSpec((RB, h3), lambda i: (i, 0)), poolspec],
        out_shape=[jax.ShapeDtypeStruct((NP, h3), jnp.float32),
                   jax.ShapeDtypeStruct((128, FP), jnp.float32)],
    )(out, ps, pq, gam, bet, ohp, w1, b1, w2, b2, w3, b3)


def _energy_body(pool_ref, w1, b1, w2, b2, w3, b3, e_ref):
    f = jnp.maximum(_dot(pool_ref[...], w1[...]) + b1[...], 0.0)
    f = jnp.maximum(_dot(f, w2[...]) + b2[...], 0.0)
    e_ref[...] = _dot(f, w3[...]) + b3[...]


def _energy_stage(pool, m2):
    (w1, b1), (w2, b2), (w3, b3) = m2
    h1, h2, h3 = w1.shape[1], w2.shape[1], w3.shape[1]
    full = lambda a: pl.BlockSpec(a.shape, lambda: tuple(0 for _ in a.shape))
    return pl.pallas_call(
        _energy_body,
        in_specs=[full(pool), full(w1), full(b1), full(w2), full(b2),
                  full(w3), full(b3)],
        out_specs=pl.BlockSpec((128, h3), lambda: (0, 0)),
        out_shape=jax.ShapeDtypeStruct((128, h3), jnp.float32),
    )(pool, w1, b1, w2, b2, w3, b3)


def _eemb_body(emb_ref, we_ref, out_ref):
    out_ref[...] = _dot(emb_ref[...], we_ref[...])


def _eemb_stage(embp, wep):
    return pl.pallas_call(
        _eemb_body,
        in_specs=[pl.BlockSpec(embp.shape, lambda: (0, 0)),
                  pl.BlockSpec(wep.shape, lambda: (0, 0))],
        out_specs=pl.BlockSpec((embp.shape[0], FP), lambda: (0, 0)),
        out_shape=jax.ShapeDtypeStruct((embp.shape[0], FP), jnp.float32),
    )(embp, wep)


def kernel(x, edge_index, edge_attr, batch, edge_emb, agg_weights,
           mlp1, pre_nn, post_nn, bn_gamma, bn_beta, mlp2, mlp3):
    xp = _pad2(x, NP, FP)
    w1p = _pad2(mlp1[0][0], FP, FP)
    b1p = _pad1(mlp1[0][1], FP)[None, :]

    w0 = pre_nn[0][0]
    wd = _pad2(w0[:F], FP, FP)
    ws = _pad2(w0[F:2 * F], FP, FP)
    we = w0[2 * F:]
    b0 = _pad1(pre_nn[0][1], FP)[None, :]

    x1, P, Q = _node_stage(xp, w1p, b1p, wd, b0, ws)

    embp = _pad2(edge_emb, 128, 16)
    wep = _pad2(we, 16, FP)
    Eemb = _eemb_stage(embp, wep)

    src = edge_index[0]
    dst = edge_index[1]
    dstp = jnp.concatenate([dst, jnp.full((EP - E,), NP - 1, jnp.int32)])
    srcp = jnp.concatenate([src, jnp.zeros((EP - E,), jnp.int32)])
    attrp = jnp.concatenate([edge_attr, jnp.zeros((EP - E,), jnp.int32)])
    perm = jnp.argsort(dstp)
    sdst = dstp[perm]
    ssrc = srcp[perm]
    sattr = attrp[perm]
    nps = NP // NW // 2
    offs = jnp.searchsorted(
        sdst, jnp.arange(2 * NW + 1, dtype=jnp.int32) * nps).astype(jnp.int32)
    offsp = jnp.pad(offs, (0, 72 - (2 * NW + 1)))

    A, B = _gather_stage(P, Q, sdst, ssrc)
    ohattr = (sattr[:, None] == jnp.arange(128)[None, :]).astype(jnp.float32)

    pre_ws = [(_pad2(w, FP, FP), _pad1(b, FP)[None, :]) for (w, b) in pre_nn[1:]]
    h = _edge_mlp(A, B, ohattr, Eemb, pre_ws)

    s, q, mn, mx, cnt128 = _segment_stage(h, sdst, offsp)
    awp = jnp.full((1, 128), -1e30, jnp.float32).at[0, :5].set(agg_weights)

    pw0 = post_nn[0][0]
    wx = _pad2(pw0[:F], FP, FP)
    wa = _pad2(pw0[F:], FP, FP)
    pb0 = _pad1(post_nn[0][1], FP)[None, :]
    post_ws = [(wx, wa, pb0)] + [(_pad2(w_, FP, FP), _pad1(b_, FP)[None, :])
                                 for (w_, b_) in post_nn[1:]]
    out, ps, pq = _post_stage(x1, s, q, mn, mx, cnt128, awp, post_ws)

    gam = _pad1(bn_gamma, FP)[None, :]
    bet = _pad1(bn_beta, FP)[None, :]
    batchp = jnp.concatenate([batch, jnp.full((NP - N,), NG, jnp.int32)])
    ohp = (batchp[:, None] == jnp.arange(128)[None, :]).astype(jnp.float32)

    def padmlp(m):
        dims = [FP] + [((w_.shape[1] + 127) // 128) * 128 for (w_, _) in m]
        return [(_pad2(w_, dims[i], dims[i + 1]),
                 _pad1(b_, dims[i + 1])[None, :]) for i, (w_, b_) in enumerate(m)]

    m3 = padmlp(mlp3)
    force_p, pool = _bn_force_stage(out, ps, pq, gam, bet, ohp, m3)
    m2 = padmlp(mlp2)
    energy_p = _energy_stage(pool, m2)

    force = force_p[:N, :3]
    energy = energy_p[:NG, :1]
    return force, energy, jnp.float32(1.0)


# speedup vs baseline: 1.4013x; 1.0137x over previous
import functools
import jax
import jax.numpy as jnp
from jax import lax
from jax.experimental import pallas as pl
from jax.experimental.pallas import tpu as pltpu
from jax.experimental.pallas import tpu_sc as plsc

F = 1262
FP = 1280
N = 10000
NP = 10240
E = 40000
EP = 40960
NG = 16
RB = 256


def _worker_id():
    return lax.axis_index("s") * 2 + lax.axis_index("c")


NW = 32
EBK = 64
FCH = 128
BIG = 3.0e38


def _pad2(a, r, c):
    return jnp.pad(a, ((0, r - a.shape[0]), (0, c - a.shape[1])))


def _pad1(a, n):
    return jnp.pad(a, ((0, n - a.shape[0]),))


def _dot(a, b):
    return jnp.dot(a, b, preferred_element_type=jnp.float32)


def _node_body(x_ref, w1, b1, wd, b0, ws, x1_out, p_out, q_out):
    x1 = jnp.maximum(_dot(x_ref[...], w1[...]) + b1[...], 0.0)
    x1_out[...] = x1
    p_out[...] = _dot(x1, wd[...]) + b0[...]
    q_out[...] = _dot(x1, ws[...])


def _node_stage(xp, w1, b1, wd, b0, ws):
    nblk = NP // RB
    full = pl.BlockSpec((FP, FP), lambda i: (0, 0))
    brow = pl.BlockSpec((1, FP), lambda i: (0, 0))
    blk = pl.BlockSpec((RB, FP), lambda i: (i, 0))
    return pl.pallas_call(
        _node_body,
        grid=(nblk,),
        in_specs=[blk, full, brow, full, brow, full],
        out_specs=[blk, blk, blk],
        out_shape=[jax.ShapeDtypeStruct((NP, FP), jnp.float32)] * 3,
    )(xp, w1, b1, wd, b0, ws)


GB = 32


def _gather_stage(P, Q, sdst, ssrc):
    epw = EP // NW
    ntl = epw // GB
    mesh = plsc.VectorSubcoreMesh(core_axis_name="c", subcore_axis_name="s")

    @functools.partial(
        pl.kernel, mesh=mesh,
        out_type=[jax.ShapeDtypeStruct((EP, FP), jnp.float32)] * 2,
        scratch_types=[
            pltpu.VMEM((GB,), jnp.int32),
            pltpu.VMEM((GB,), jnp.int32),
            pltpu.VMEM((GB, FP), jnp.float32),
            pltpu.VMEM((GB, FP), jnp.float32),
            pltpu.SemaphoreType.DMA,
        ],
    )
    def gk(p_hbm, q_hbm, d_hbm, s_hbm, a_hbm, b_hbm,
           idx_d, idx_s, buf_a, buf_b, sem):
        wid = _worker_id()
        base = wid * epw

        def tile(t, _):
            e0 = base + t * GB
            pltpu.sync_copy(d_hbm.at[pl.ds(e0, GB)], idx_d)
            pltpu.sync_copy(s_hbm.at[pl.ds(e0, GB)], idx_s)
            ca = pltpu.async_copy(p_hbm.at[idx_d], buf_a, sem)
            cb = pltpu.async_copy(q_hbm.at[idx_s], buf_b, sem)
            ca.wait()
            cb.wait()
            pltpu.sync_copy(buf_a, a_hbm.at[pl.ds(e0, GB)])
            pltpu.sync_copy(buf_b, b_hbm.at[pl.ds(e0, GB)])
            return 0
        lax.fori_loop(0, ntl, tile, 0)

    return gk(P, Q, sdst, ssrc)


def _edge_mlp_body(a_ref, b_ref, oh_ref, eemb, w1, b1, w2, b2, w3, b3,
                   w4, b4, h_out):
    e = jnp.dot(oh_ref[...], eemb[...], preferred_element_type=jnp.float32,
                precision=jax.lax.Precision.HIGHEST)
    h = jnp.maximum(a_ref[...] + b_ref[...] + e, 0.0)
    h = jnp.maximum(_dot(h, w1[...]) + b1[...], 0.0)
    h = jnp.maximum(_dot(h, w2[...]) + b2[...], 0.0)
    h = jnp.maximum(_dot(h, w3[...]) + b3[...], 0.0)
    h_out[...] = _dot(h, w4[...]) + b4[...]


def _edge_mlp(a, b, oh, eemb, ws):
    nblk = EP // RB
    full = pl.BlockSpec((FP, FP), lambda i: (0, 0))
    brow = pl.BlockSpec((1, FP), lambda i: (0, 0))
    blk = pl.BlockSpec((RB, FP), lambda i: (i, 0))
    ohspec = pl.BlockSpec((RB, 128), lambda i: (i, 0))
    embspec = pl.BlockSpec((128, FP), lambda i: (0, 0))
    args = []
    for (w, b_) in ws:
        args += [w, b_]
    return pl.pallas_call(
        _edge_mlp_body,
        grid=(nblk,),
        in_specs=[blk, blk, ohspec, embspec] + [full, brow] * 4,
        out_specs=blk,
        out_shape=jax.ShapeDtypeStruct((EP, FP), jnp.float32),
    )(a, b, oh, eemb, *args)


def _segment_stage(h, sdst, offs):
    npw = NP // NW
    nsb = 2
    nps = npw // nsb
    nch = FP // FCH
    nsl = FCH // 16
    mesh = plsc.VectorSubcoreMesh(core_axis_name="c", subcore_axis_name="s")

    @functools.partial(
        pl.kernel, mesh=mesh,
        out_type=[jax.ShapeDtypeStruct((NP, FP), jnp.float32)] * 4
        + [jax.ShapeDtypeStruct((NP, FCH), jnp.float32)],
        scratch_types=[
            pltpu.VMEM((80,), jnp.int32),
            pltpu.VMEM((EBK + 16,), jnp.int32),
            pltpu.VMEM((EBK, FCH), jnp.float32),
            pltpu.VMEM((NP // NW // 2, FCH), jnp.float32),
            pltpu.VMEM((NP // NW // 2, FCH), jnp.float32),
            pltpu.VMEM((NP // NW // 2, FCH), jnp.float32),
            pltpu.VMEM((NP // NW // 2, FCH), jnp.float32),
        ],
    )
    def seg(h_hbm, dst_hbm, offs_hbm, s_hbm, q_hbm, mn_hbm, mx_hbm, c_hbm,
            offs_v, dst_v, hbuf, acc_s, acc_q, acc_mn, acc_mx):
        wid = _worker_id()
        pltpu.sync_copy(offs_hbm, offs_v.at[pl.ds(0, 72)])
        ov = offs_v[pl.ds(nsb * wid, 16)]

        for sb in range(nsb):
            node0 = wid * npw + sb * nps
            lo = ov[sb]
            hi = ov[sb + 1]
            t0 = lo // EBK
            t1 = (hi + EBK - 1) // EBK

            def zc(i, _):
                z = jnp.zeros((16,), jnp.float32)
                for kk in range(nsl):
                    acc_mx[i, pl.ds(kk * 16, 16)] = z
                return 0
            lax.fori_loop(0, nps, zc, 0, unroll=8)

            def cnt_tile(t, _):
                e0 = t * EBK
                pltpu.sync_copy(dst_hbm.at[pl.ds(e0, EBK)],
                                dst_v.at[pl.ds(0, EBK)])

                def edge(e, __):
                    eg = e0 + e

                    @pl.when(jnp.logical_and(eg >= lo, eg < hi))
                    def _():
                        n = dst_v[pl.ds(e, 16)][0] - node0
                        for kk in range(nsl):
                            sl = pl.ds(kk * 16, 16)
                            acc_mx[n, sl] = acc_mx[n, sl] + 1.0
                    return 0
                lax.fori_loop(0, EBK, edge, 0)
                return 0
            lax.fori_loop(t0, t1, cnt_tile, 0)
            pltpu.sync_copy(acc_mx, c_hbm.at[pl.ds(node0, nps)])

            def chunk(ci, _):
                c0 = ci * FCH

                def zi(i, __):
                    z = jnp.zeros((16,), jnp.float32)
                    rmn = jnp.full((16,), BIG, jnp.float32)
                    rmx = jnp.full((16,), -BIG, jnp.float32)
                    for kk in range(nsl):
                        sl = pl.ds(kk * 16, 16)
                        acc_s[i, sl] = z
                        acc_q[i, sl] = z
                        acc_mn[i, sl] = rmn
                        acc_mx[i, sl] = rmx
                    return 0
                lax.fori_loop(0, nps, zi, 0, unroll=4)

                def tile(t, __):
                    e0 = t * EBK
                    pltpu.sync_copy(dst_hbm.at[pl.ds(e0, EBK)],
                                    dst_v.at[pl.ds(0, EBK)])
                    pltpu.sync_copy(h_hbm.at[pl.ds(e0, EBK), pl.ds(c0, FCH)],
                                    hbuf)

                    def edge(e, ___):
                        eg = e0 + e

                        @pl.when(jnp.logical_and(eg >= lo, eg < hi))
                        def _():
                            n = dst_v[pl.ds(e, 16)][0] - node0
                            for kk in range(nsl):
                                sl = pl.ds(kk * 16, 16)
                                hv = hbuf[e, sl]
                                acc_s[n, sl] = acc_s[n, sl] + hv
                                acc_q[n, sl] = acc_q[n, sl] + hv * hv
                                acc_mn[n, sl] = jnp.minimum(acc_mn[n, sl], hv)
                                acc_mx[n, sl] = jnp.maximum(acc_mx[n, sl], hv)
                        return 0
                    lax.fori_loop(0, EBK, edge, 0)
                    return 0
                lax.fori_loop(t0, t1, tile, 0)
                pltpu.sync_copy(acc_s, s_hbm.at[pl.ds(node0, nps), pl.ds(c0, FCH)])
                pltpu.sync_copy(acc_q, q_hbm.at[pl.ds(node0, nps), pl.ds(c0, FCH)])
                pltpu.sync_copy(acc_mn, mn_hbm.at[pl.ds(node0, nps), pl.ds(c0, FCH)])
                pltpu.sync_copy(acc_mx, mx_hbm.at[pl.ds(node0, nps), pl.ds(c0, FCH)])
                return 0
            lax.fori_loop(0, nch, chunk, 0)

    return seg(h, sdst, offs)


def _post_body(x1_ref, s_ref, q_ref, mn_ref, mx_ref, cnt_ref, aw_ref,
               wx, wa, b0, w1, b1, w2, b2, w3, b3, w4, b4,
               out_ref, ps_ref, pq_ref):
    i = pl.program_id(0)
    awv = aw_ref[...]
    ex = jnp.exp(awv - jnp.max(awv))
    sm = ex / jnp.sum(ex)
    lane = jax.lax.broadcasted_iota(jnp.int32, (1, 128), 1)
    wk = [jnp.sum(jnp.where(lane == k, sm, 0.0)) for k in range(5)]
    cnt = cnt_ref[...][:, :1]
    pos = cnt > 0.0
    s = jnp.where(pos, s_ref[...], 0.0)
    q = jnp.where(pos, q_ref[...], 0.0)
    mn = jnp.where(pos, mn_ref[...], 0.0)
    mx = jnp.where(pos, mx_ref[...], 0.0)
    r = 1.0 / jnp.maximum(cnt, 1.0)
    mean = s * r
    std = jnp.sqrt(jnp.maximum(q * r - mean * mean, 0.0) + 1e-5)
    agg = wk[0] * s + wk[1] * mean + wk[2] * mn + wk[3] * mx + wk[4] * std
    h = _dot(x1_ref[...], wx[...]) + _dot(agg, wa[...]) + b0[...]
    h = jnp.maximum(h, 0.0)
    h = jnp.maximum(_dot(h, w1[...]) + b1[...], 0.0)
    h = jnp.maximum(_dot(h, w2[...]) + b2[...], 0.0)
    h = jnp.maximum(_dot(h, w3[...]) + b3[...], 0.0)
    h = _dot(h, w4[...]) + b4[...]
    out_ref[...] = h
    rows = jax.lax.broadcasted_iota(jnp.int32, (RB, 1), 0) + i * RB
    m = (rows < N).astype(jnp.float32)
    hm = h * m
    ps = jnp.sum(hm.reshape(RB // 8, 8, FP), axis=0)
    pq = jnp.sum((hm * hm).reshape(RB // 8, 8, FP), axis=0)

    @pl.when(i == 0)
    def _():
        ps_ref[...] = jnp.zeros_like(ps_ref)
        pq_ref[...] = jnp.zeros_like(pq_ref)

    ps_ref[...] += ps
    pq_ref[...] += pq


def _post_stage(x1, s, q, mn, mx, cnt128, awp, ws):
    nblk = NP // RB
    full = pl.BlockSpec((FP, FP), lambda i: (0, 0))
    brow = pl.BlockSpec((1, FP), lambda i: (0, 0))
    brow128 = pl.BlockSpec((1, 128), lambda i: (0, 0))
    blk = pl.BlockSpec((RB, FP), lambda i: (i, 0))
    blk128 = pl.BlockSpec((RB, FCH), lambda i: (i, 0))
    acc = pl.BlockSpec((8, FP), lambda i: (0, 0))
    args = []
    for (w, b) in ws[1:]:
        args += [w, b]
    return pl.pallas_call(
        _post_body,
        grid=(nblk,),
        in_specs=[blk, blk, blk, blk, blk, blk128, brow128,
                  full, full, brow] + [full, brow] * 4,
        out_specs=[blk, acc, acc],
        out_shape=[jax.ShapeDtypeStruct((NP, FP), jnp.float32),
                   jax.ShapeDtypeStruct((8, FP), jnp.float32),
                   jax.ShapeDtypeStruct((8, FP), jnp.float32)],
    )(x1, s, q, mn, mx, cnt128, awp, ws[0][0], ws[0][1], ws[0][2], *args)


def _bn_force_body(out_ref, ps_ref, pq_ref, gam, bet, oh_ref,
                   w1, b1, w2, b2, w3, b3, force_ref, pool_ref):
    i = pl.program_id(0)
    mu = jnp.sum(ps_ref[...], axis=0, keepdims=True) / N
    var = jnp.sum(pq_ref[...], axis=0, keepdims=True) / N - mu * mu
    h = (out_ref[...] - mu) * jax.lax.rsqrt(var + 1e-5) * gam[...] + bet[...]
    h = jnp.maximum(h, 0.0)
    part = jax.lax.dot_general(oh_ref[...], h, (((0,), (0,)), ((), ())),
                               preferred_element_type=jnp.float32)

    @pl.when(i == 0)
    def _():
        pool_ref[...] = jnp.zeros_like(pool_ref)

    pool_ref[...] += part
    f = jnp.maximum(_dot(h, w1[...]) + b1[...], 0.0)
    f = jnp.maximum(_dot(f, w2[...]) + b2[...], 0.0)
    force_ref[...] = _dot(f, w3[...]) + b3[...]


def _bn_force_stage(out, ps, pq, gam, bet, ohp, m3):
    nblk = NP // RB
    blk = pl.BlockSpec((RB, FP), lambda i: (i, 0))
    acc8 = pl.BlockSpec((8, FP), lambda i: (0, 0))
    brow = pl.BlockSpec((1, FP), lambda i: (0, 0))
    bblk = pl.BlockSpec((RB, 128), lambda i: (i, 0))
    poolspec = pl.BlockSpec((128, FP), lambda i: (0, 0))
    (w1, b1), (w2, b2), (w3, b3) = m3
    h1, h2, h3 = w1.shape[1], w2.shape[1], w3.shape[1]
    specs = [blk, acc8, acc8, brow, brow, bblk,
             pl.BlockSpec((FP, h1), lambda i: (0, 0)),
             pl.BlockSpec((1, h1), lambda i: (0, 0)),
             pl.BlockSpec((h1, h2), lambda i: (0, 0)),
             pl.BlockSpec((1, h2), lambda i: (0, 0)),
             pl.BlockSpec((h2, h3), lambda i: (0, 0)),
             pl.BlockSpec((1, h3), lambda i: (0, 0))]
    return pl.pallas_call(
        _bn_force_body,
        grid=(nblk,),
        in_specs=specs,
        out_specs=[pl.BlockSpec((RB, h3), lambda i: (i, 0)), poolspec],
        out_shape=[jax.ShapeDtypeStruct((NP, h3), jnp.float32),
                   jax.ShapeDtypeStruct((128, FP), jnp.float32)],
    )(out, ps, pq, gam, bet, ohp, w1, b1, w2, b2, w3, b3)


def _energy_body(pool_ref, w1, b1, w2, b2, w3, b3, e_ref):
    f = jnp.maximum(_dot(pool_ref[...], w1[...]) + b1[...], 0.0)
    f = jnp.maximum(_dot(f, w2[...]) + b2[...], 0.0)
    e_ref[...] = _dot(f, w3[...]) + b3[...]


def _energy_stage(pool, m2):
    (w1, b1), (w2, b2), (w3, b3) = m2
    h1, h2, h3 = w1.shape[1], w2.shape[1], w3.shape[1]
    full = lambda a: pl.BlockSpec(a.shape, lambda: tuple(0 for _ in a.shape))
    return pl.pallas_call(
        _energy_body,
        in_specs=[full(pool), full(w1), full(b1), full(w2), full(b2),
                  full(w3), full(b3)],
        out_specs=pl.BlockSpec((128, h3), lambda: (0, 0)),
        out_shape=jax.ShapeDtypeStruct((128, h3), jnp.float32),
    )(pool, w1, b1, w2, b2, w3, b3)


def _eemb_body(emb_ref, we_ref, out_ref):
    out_ref[...] = _dot(emb_ref[...], we_ref[...])


def _eemb_stage(embp, wep):
    return pl.pallas_call(
        _eemb_body,
        in_specs=[pl.BlockSpec(embp.shape, lambda: (0, 0)),
                  pl.BlockSpec(wep.shape, lambda: (0, 0))],
        out_specs=pl.BlockSpec((embp.shape[0], FP), lambda: (0, 0)),
        out_shape=jax.ShapeDtypeStruct((embp.shape[0], FP), jnp.float32),
    )(embp, wep)


def kernel(x, edge_index, edge_attr, batch, edge_emb, agg_weights,
           mlp1, pre_nn, post_nn, bn_gamma, bn_beta, mlp2, mlp3):
    xp = _pad2(x, NP, FP)
    w1p = _pad2(mlp1[0][0], FP, FP)
    b1p = _pad1(mlp1[0][1], FP)[None, :]

    w0 = pre_nn[0][0]
    wd = _pad2(w0[:F], FP, FP)
    ws = _pad2(w0[F:2 * F], FP, FP)
    we = w0[2 * F:]
    b0 = _pad1(pre_nn[0][1], FP)[None, :]

    x1, P, Q = _node_stage(xp, w1p, b1p, wd, b0, ws)

    embp = _pad2(edge_emb, 128, 16)
    wep = _pad2(we, 16, FP)
    Eemb = _eemb_stage(embp, wep)

    src = edge_index[0]
    dst = edge_index[1]
    dstp = jnp.concatenate([dst, jnp.full((EP - E,), NP - 1, jnp.int32)])
    srcp = jnp.concatenate([src, jnp.zeros((EP - E,), jnp.int32)])
    attrp = jnp.concatenate([edge_attr, jnp.zeros((EP - E,), jnp.int32)])
    perm = jnp.argsort(dstp)
    sdst = dstp[perm]
    ssrc = srcp[perm]
    sattr = attrp[perm]
    nps = NP // NW // 2
    offs = jnp.searchsorted(
        sdst, jnp.arange(2 * NW + 1, dtype=jnp.int32) * nps).astype(jnp.int32)
    offsp = jnp.pad(offs, (0, 72 - (2 * NW + 1)))

    A, B = _gather_stage(P, Q, sdst, ssrc)
    ohattr = (sattr[:, None] == jnp.arange(128)[None, :]).astype(jnp.float32)

    pre_ws = [(_pad2(w, FP, FP), _pad1(b, FP)[None, :]) for (w, b) in pre_nn[1:]]
    h = _edge_mlp(A, B, ohattr, Eemb, pre_ws)

    s, q, mn, mx, cnt128 = _segment_stage(h, sdst, offsp)
    awp = jnp.full((1, 128), -1e30, jnp.float32).at[0, :5].set(agg_weights)

    pw0 = post_nn[0][0]
    wx = _pad2(pw0[:F], FP, FP)
    wa = _pad2(pw0[F:], FP, FP)
    pb0 = _pad1(post_nn[0][1], FP)[None, :]
    post_ws = [(wx, wa, pb0)] + [(_pad2(w_, FP, FP), _pad1(b_, FP)[None, :])
                                 for (w_, b_) in post_nn[1:]]
    out, ps, pq = _post_stage(x1, s, q, mn, mx, cnt128, awp, post_ws)

    gam = _pad1(bn_gamma, FP)[None, :]
    bet = _pad1(bn_beta, FP)[None, :]
    batchp = jnp.concatenate([batch, jnp.full((NP - N,), NG, jnp.int32)])
    ohp = (batchp[:, None] == jnp.arange(128)[None, :]).astype(jnp.float32)

    def padmlp(m):
        dims = [FP] + [((w_.shape[1] + 127) // 128) * 128 for (w_, _) in m]
        return [(_pad2(w_, dims[i], dims[i + 1]),
                 _pad1(b_, dims[i + 1])[None, :]) for i, (w_, b_) in enumerate(m)]

    m3 = padmlp(mlp3)
    force_p, pool = _bn_force_stage(out, ps, pq, gam, bet, ohp, m3)
    m2 = padmlp(mlp2)
    energy_p = _energy_stage(pool, m2)

    force = force_p[:N, :3]
    energy = energy_p[:NG, :1]
    return force, energy, jnp.float32(1.0)
